# baseline (speedup 1.0000x reference)
import functools

import jax
import jax.numpy as jnp
from jax import lax
from jax.experimental import pallas as pl
from jax.experimental.pallas import tpu as pltpu
from jax.experimental.pallas import tpu_sc as plsc

_VOCAB = 1000000
_DIM = 64
_BATCH = 16384
_NEG = 20

_CHUNK = 128


def _sc_gather(in_emb, out_emb, tw, cw, nwf):
    info = plsc.get_sparse_core_info()
    nworkers = info.num_cores * info.num_subcores
    t_chunks = _BATCH // (nworkers * _CHUNK)
    n_chunks = _BATCH * _NEG // (nworkers * _CHUNK)

    mesh = plsc.VectorSubcoreMesh(core_axis_name="c", subcore_axis_name="s")

    @functools.partial(
        pl.kernel,
        mesh=mesh,
        out_type=(
            jax.ShapeDtypeStruct((_BATCH, _DIM), jnp.float32),
            jax.ShapeDtypeStruct((_BATCH, _DIM), jnp.float32),
            jax.ShapeDtypeStruct((_BATCH * _NEG, _DIM), jnp.float32),
        ),
        scratch_types=[
            pltpu.VMEM((_CHUNK,), jnp.int32),
            pltpu.VMEM((_CHUNK, _DIM), jnp.float32),
            pltpu.SemaphoreType.DMA,
            pltpu.SemaphoreType.DMA,
        ],
    )
    def k(in_hbm, out_hbm, tw_hbm, cw_hbm, nw_hbm,
          tgt_hbm, ctx_hbm, neg_hbm,
          idx_v, buf, gsem, osem):
        wid = lax.axis_index("s") * info.num_cores + lax.axis_index("c")

        def gather_chunks(table, idx_hbm, idx_base, chunks, dst):
            def chunk_body(jc, _):
                row0 = (idx_base + jc) * _CHUNK
                pltpu.sync_copy(idx_hbm.at[pl.ds(row0, _CHUNK)], idx_v)

                def group_body(g, _):
                    idxv = idx_v[pl.ds(g * 16, 16)]
                    for r in range(16):
                        i = idxv[r]
                        pltpu.async_copy(table.at[pl.ds(i, 1)],
                                         buf.at[pl.ds(g * 16 + r, 1)], gsem)
                    return 0

                lax.fori_loop(0, _CHUNK // 16, group_body, 0)
                pltpu.make_async_copy(table.at[pl.ds(0, _CHUNK)], buf,
                                      gsem).wait()
                pltpu.async_copy(buf, dst.at[pl.ds(row0, _CHUNK)],
                                 osem).wait()
                return 0

            lax.fori_loop(0, chunks, chunk_body, 0)

        gather_chunks(in_hbm, tw_hbm, wid * t_chunks, t_chunks, tgt_hbm)
        gather_chunks(out_hbm, cw_hbm, wid * t_chunks, t_chunks, ctx_hbm)
        gather_chunks(out_hbm, nw_hbm, wid * n_chunks, n_chunks, neg_hbm)

    return k(in_emb, out_emb, tw, cw, nwf)


def _tc_loss_body(t_ref, c_ref, n_ref, lt_ref, o_ref):
    i = pl.program_id(0)
    t = t_ref[...]
    c = c_ref[...]
    n = n_ref[...]
    inv_temp = jnp.exp(-lt_ref[0])

    tn = jnp.maximum(jnp.sqrt(jnp.sum(t * t, axis=1, keepdims=True)), 1e-12)
    cn = jnp.maximum(jnp.sqrt(jnp.sum(c * c, axis=1, keepdims=True)), 1e-12)
    nn = jnp.maximum(jnp.sqrt(jnp.sum(n * n, axis=2)), 1e-12)

    pos = jnp.sum(t * c, axis=1, keepdims=True) / (tn * cn) * inv_temp
    negs = jnp.sum(n * t[:, None, :], axis=2) / (nn * tn) * inv_temp

    m = jnp.maximum(pos[:, 0], jnp.max(negs, axis=1))
    lse = m + jnp.log(jnp.exp(pos[:, 0] - m)
                      + jnp.sum(jnp.exp(negs - m[:, None]), axis=1))
    part = jnp.sum(lse - pos[:, 0]) * (1.0 / _BATCH)

    @pl.when(i == 0)
    def _():
        o_ref[0, 0] = 0.0

    o_ref[0, 0] += part


def _tc_loss(tgt, ctx, neg3, log_temperature):
    bs = 512
    grid = _BATCH // bs
    out = pl.pallas_call(
        _tc_loss_body,
        grid=(grid,),
        in_specs=[
            pl.BlockSpec((bs, _DIM), lambda i: (i, 0)),
            pl.BlockSpec((bs, _DIM), lambda i: (i, 0)),
            pl.BlockSpec((bs, _NEG, _DIM), lambda i: (i, 0, 0)),
            pl.BlockSpec(memory_space=pltpu.SMEM),
        ],
        out_specs=pl.BlockSpec(memory_space=pltpu.SMEM),
        out_shape=jax.ShapeDtypeStruct((1, 1), jnp.float32),
    )(tgt, ctx, neg3, jnp.reshape(log_temperature, (1,)))
    return out[0, 0]


def kernel(target_words, context_words, neg_words, in_emb, out_emb,
           log_temperature):
    tw = target_words.astype(jnp.int32)
    cw = context_words.astype(jnp.int32)
    nwf = jnp.reshape(neg_words.astype(jnp.int32), (-1,))

    tgt, ctx, neg = _sc_gather(in_emb, out_emb, tw, cw, nwf)
    neg3 = jnp.reshape(neg, (_BATCH, _NEG, _DIM))
    return _tc_loss(tgt, ctx, neg3, log_temperature)


# speedup vs baseline: 1.4785x; 1.4785x over previous
import functools

import jax
import jax.numpy as jnp
from jax import lax
from jax.experimental import pallas as pl
from jax.experimental.pallas import tpu as pltpu
from jax.experimental.pallas import tpu_sc as plsc

_VOCAB = 1000000
_DIM = 64
_BATCH = 16384
_NEG = 20

_BG = 16
_RPB = _BG * (2 + _NEG)

_ROW_POS = 0
_ROW_TN = 1
_ROW_CN = 2
_ROW_ND = 8
_ROW_NN = 32
_NROWS = 64


def _sc_dots(in_emb, out_emb, tw, cw, nwf):
    info = plsc.get_sparse_core_info()
    nworkers = info.num_cores * info.num_subcores
    b_per_w = _BATCH // nworkers
    nblk = b_per_w // _BG

    mesh = plsc.VectorSubcoreMesh(core_axis_name="c", subcore_axis_name="s")

    @functools.partial(
        pl.kernel,
        mesh=mesh,
        compiler_params=pltpu.CompilerParams(needs_layout_passes=False),
        out_type=jax.ShapeDtypeStruct((_NROWS, _BATCH), jnp.float32),
        scratch_types=[
            pltpu.VMEM((2 * _RPB,), jnp.int32),
            pltpu.VMEM((2, _RPB, _DIM), jnp.float32),
            pltpu.VMEM((_NROWS, b_per_w), jnp.float32),
            pltpu.SemaphoreType.DMA,
            pltpu.SemaphoreType.DMA,
            pltpu.SemaphoreType.DMA,
        ],
    )
    def k(in_hbm, out_hbm, tw_hbm, cw_hbm, nw_hbm, out_p,
          idx_v, buf, stage, isem, gsem, osem):
        wid = lax.axis_index("s") * info.num_cores + lax.axis_index("c")
        b0w = wid * b_per_w

        lane = lax.iota(jnp.int32, 16)
        m15 = lane == 15

        def issue_idx(blk, slot):
            b0 = b0w + blk * _BG
            off = slot * _RPB
            pltpu.async_copy(tw_hbm.at[pl.ds(b0, _BG)],
                             idx_v.at[pl.ds(off, _BG)], isem)
            pltpu.async_copy(cw_hbm.at[pl.ds(b0, _BG)],
                             idx_v.at[pl.ds(off + _BG, _BG)], isem)
            pltpu.async_copy(nw_hbm.at[pl.ds(b0 * _NEG, _BG * _NEG)],
                             idx_v.at[pl.ds(off + 2 * _BG, _BG * _NEG)],
                             isem)

        def wait_idx():
            pltpu.make_async_copy(tw_hbm.at[pl.ds(0, _RPB)],
                                  idx_v.at[pl.ds(0, _RPB)], isem).wait()

        def issue_rows(slot):
            def group(g, table, _):
                idxv = idx_v[pl.ds(slot * _RPB + g * 16, 16)]
                for r in range(16):
                    pltpu.async_copy(
                        table.at[pl.ds(idxv[r], 1)],
                        buf.at[slot, pl.ds(g * 16 + r, 1)], gsem)
                return 0

            group(0, in_hbm, 0)
            lax.fori_loop(1, _RPB // 16,
                          lambda g, c: group(g, out_hbm, c), 0)

        def wait_rows():
            pltpu.make_async_copy(in_hbm.at[pl.ds(0, _RPB)],
                                  buf.at[0], gsem).wait()

        def ld(slot, row, j):
            return buf[slot, row, pl.ds(j * 16, 16)]

        def dot4(a, b):
            s = a[0] * b[0] + a[1] * b[1]
            s = s + a[2] * b[2] + a[3] * b[3]
            return plsc.cumsum(s)

        def put(row, p, v):
            plsc.store_scatter(stage,
                               [jnp.full((16,), row, jnp.int32),
                                jnp.full((16,), p, jnp.int32)],
                               v, mask=m15)

        def compute(blk, slot):
            def body(lb, _):
                p = blk * _BG + lb
                t = [ld(slot, lb, j) for j in range(4)]
                c = [ld(slot, _BG + lb, j) for j in range(4)]
                put(_ROW_TN, p, dot4(t, t))
                put(_ROW_CN, p, dot4(c, c))
                put(_ROW_POS, p, dot4(t, c))
                for kk in range(_NEG):
                    row = 2 * _BG + lb * _NEG + kk
                    n = [ld(slot, row, j) for j in range(4)]
                    put(_ROW_ND + kk, p, dot4(n, t))
                    put(_ROW_NN + kk, p, dot4(n, n))
                return 0

            lax.fori_loop(0, _BG, body, 0)

        issue_idx(0, 0)
        wait_idx()
        issue_rows(0)
        issue_idx(1, 1)

        def blk_body(blk, _):
            slot = lax.rem(blk, 2)
            nslot = lax.rem(blk + 1, 2)

            wait_rows()

            @pl.when(blk + 1 < nblk)
            def _():
                wait_idx()
                issue_rows(nslot)

            @pl.when(blk + 2 < nblk)
            def _():
                issue_idx(blk + 2, slot)

            compute(blk, slot)
            return 0

        lax.fori_loop(0, nblk, blk_body, 0)

        pltpu.async_copy(stage, out_p.at[:, pl.ds(b0w, b_per_w)],
                         osem).wait()

    return k(in_emb, out_emb, tw, cw, nwf)


def _tc_loss_body(x_ref, lt_ref, o_ref):
    i = pl.program_id(0)
    inv_temp = jnp.exp(-lt_ref[0])

    pos_d = x_ref[_ROW_POS:_ROW_POS + 1, :]
    tn = x_ref[_ROW_TN:_ROW_TN + 1, :]
    cn = x_ref[_ROW_CN:_ROW_CN + 1, :]
    nd = x_ref[_ROW_ND:_ROW_ND + _NEG, :]
    nn = x_ref[_ROW_NN:_ROW_NN + _NEG, :]

    tnorm = jnp.maximum(jnp.sqrt(tn), 1e-12)
    cnorm = jnp.maximum(jnp.sqrt(cn), 1e-12)
    nnorm = jnp.maximum(jnp.sqrt(nn), 1e-12)

    pos = pos_d / (tnorm * cnorm) * inv_temp
    negs = nd / (nnorm * tnorm) * inv_temp

    m = jnp.maximum(pos, jnp.max(negs, axis=0, keepdims=True))
    lse = m + jnp.log(jnp.exp(pos - m)
                      + jnp.sum(jnp.exp(negs - m), axis=0, keepdims=True))
    part = jnp.sum(lse - pos) * (1.0 / _BATCH)

    @pl.when(i == 0)
    def _():
        o_ref[0, 0] = 0.0

    o_ref[0, 0] += part


def _tc_loss(partials, log_temperature):
    bs = 2048
    grid = _BATCH // bs
    out = pl.pallas_call(
        _tc_loss_body,
        grid=(grid,),
        in_specs=[
            pl.BlockSpec((_NROWS, bs), lambda i: (0, i)),
            pl.BlockSpec(memory_space=pltpu.SMEM),
        ],
        out_specs=pl.BlockSpec(memory_space=pltpu.SMEM),
        out_shape=jax.ShapeDtypeStruct((1, 1), jnp.float32),
    )(partials, jnp.reshape(log_temperature, (1,)))
    return out[0, 0]


def kernel(target_words, context_words, neg_words, in_emb, out_emb,
           log_temperature):
    tw = target_words.astype(jnp.int32)
    cw = context_words.astype(jnp.int32)
    nwf = jnp.reshape(neg_words.astype(jnp.int32), (-1,))

    partials = _sc_dots(in_emb, out_emb, tw, cw, nwf)
    return _tc_loss(partials, log_temperature)


# speedup vs baseline: 1.6757x; 1.1334x over previous
import functools

import jax
import jax.numpy as jnp
from jax import lax
from jax.experimental import pallas as pl
from jax.experimental.pallas import tpu as pltpu
from jax.experimental.pallas import tpu_sc as plsc

_VOCAB = 1000000
_DIM = 64
_BATCH = 16384
_NEG = 20

_BG = 16
_RPB = _BG * (1 + _NEG)

_ROW_POS = 0
_ROW_TN = 1
_ROW_CN = 2
_ROW_ND = 8
_ROW_NN = 32
_NROWS = 64


def _sc_dots(trows, out_emb, cw, nwf):
    info = plsc.get_sparse_core_info()
    nworkers = info.num_cores * info.num_subcores
    b_per_w = _BATCH // nworkers
    nblk = b_per_w // _BG

    mesh = plsc.VectorSubcoreMesh(core_axis_name="c", subcore_axis_name="s")

    @functools.partial(
        pl.kernel,
        mesh=mesh,
        compiler_params=pltpu.CompilerParams(needs_layout_passes=False),
        out_type=jax.ShapeDtypeStruct((_NROWS, _BATCH), jnp.float32),
        scratch_types=[
            pltpu.VMEM((2 * _RPB,), jnp.int32),
            pltpu.VMEM((2, _RPB, _DIM), jnp.float32),
            pltpu.VMEM((2, _BG, _DIM), jnp.float32),
            pltpu.VMEM((_NROWS, b_per_w), jnp.float32),
            pltpu.SemaphoreType.DMA,
            pltpu.SemaphoreType.DMA,
            pltpu.SemaphoreType.DMA,
            pltpu.SemaphoreType.DMA,
        ],
    )
    def k(t_hbm, out_hbm, cw_hbm, nw_hbm, out_p,
          idx_v, buf, tcol, stage, isem, gsem, tsem, osem):
        wid = lax.axis_index("s") * info.num_cores + lax.axis_index("c")
        b0w = wid * b_per_w

        lane = lax.iota(jnp.int32, 16)
        m15 = lane == 15

        def issue_idx(blk, slot):
            b0 = b0w + blk * _BG
            off = slot * _RPB
            pltpu.async_copy(cw_hbm.at[pl.ds(b0, _BG)],
                             idx_v.at[pl.ds(off, _BG)], isem)
            pltpu.async_copy(nw_hbm.at[pl.ds(b0 * _NEG, _BG * _NEG)],
                             idx_v.at[pl.ds(off + _BG, _BG * _NEG)],
                             isem)

        def wait_idx():
            pltpu.make_async_copy(cw_hbm.at[pl.ds(0, _RPB)],
                                  idx_v.at[pl.ds(0, _RPB)], isem).wait()

        def issue_rows(blk, slot):
            b0 = b0w + blk * _BG
            pltpu.async_copy(t_hbm.at[pl.ds(b0, _BG)], tcol.at[slot], tsem)

            def group(g, _):
                idxv = idx_v[pl.ds(slot * _RPB + g * 16, 16)]
                for r in range(16):
                    pltpu.async_copy(
                        out_hbm.at[pl.ds(idxv[r], 1)],
                        buf.at[slot, pl.ds(g * 16 + r, 1)], gsem)
                return 0

            lax.fori_loop(0, _RPB // 16, group, 0)

        def wait_rows():
            pltpu.make_async_copy(t_hbm.at[pl.ds(0, _BG)],
                                  tcol.at[0], tsem).wait()
            pltpu.make_async_copy(out_hbm.at[pl.ds(0, _RPB)],
                                  buf.at[0], gsem).wait()

        def dot4(a, b):
            s = a[0] * b[0] + a[1] * b[1]
            s = s + a[2] * b[2] + a[3] * b[3]
            return plsc.cumsum(s)

        def put(row, p, v):
            plsc.store_scatter(stage,
                               [jnp.full((16,), row, jnp.int32),
                                jnp.full((16,), p, jnp.int32)],
                               v, mask=m15)

        def compute(blk, slot):
            def body(lb, _):
                p = blk * _BG + lb
                t = [tcol[slot, lb, pl.ds(j * 16, 16)] for j in range(4)]
                c = [buf[slot, lb, pl.ds(j * 16, 16)] for j in range(4)]
                put(_ROW_TN, p, dot4(t, t))
                put(_ROW_CN, p, dot4(c, c))
                put(_ROW_POS, p, dot4(t, c))
                for kk in range(_NEG):
                    row = _BG + lb * _NEG + kk
                    n = [buf[slot, row, pl.ds(j * 16, 16)]
                         for j in range(4)]
                    put(_ROW_ND + kk, p, dot4(n, t))
                    put(_ROW_NN + kk, p, dot4(n, n))
                return 0

            lax.fori_loop(0, _BG, body, 0)

        issue_idx(0, 0)
        wait_idx()
        issue_rows(0, 0)
        issue_idx(1, 1)

        def blk_body(blk, _):
            slot = lax.rem(blk, 2)
            nslot = lax.rem(blk + 1, 2)

            wait_rows()

            @pl.when(blk + 1 < nblk)
            def _():
                wait_idx()
                issue_rows(blk + 1, nslot)

            @pl.when(blk + 2 < nblk)
            def _():
                issue_idx(blk + 2, slot)

            compute(blk, slot)
            return 0

        lax.fori_loop(0, nblk, blk_body, 0)

        pltpu.async_copy(stage, out_p.at[:, pl.ds(b0w, b_per_w)],
                         osem).wait()

    return k(trows, out_emb, cw, nwf)


def _tc_loss_body(x_ref, lt_ref, o_ref):
    i = pl.program_id(0)
    inv_temp = jnp.exp(-lt_ref[0])

    pos_d = x_ref[_ROW_POS:_ROW_POS + 1, :]
    tn = x_ref[_ROW_TN:_ROW_TN + 1, :]
    cn = x_ref[_ROW_CN:_ROW_CN + 1, :]
    nd = x_ref[_ROW_ND:_ROW_ND + _NEG, :]
    nn = x_ref[_ROW_NN:_ROW_NN + _NEG, :]

    tnorm = jnp.maximum(jnp.sqrt(tn), 1e-12)
    cnorm = jnp.maximum(jnp.sqrt(cn), 1e-12)
    nnorm = jnp.maximum(jnp.sqrt(nn), 1e-12)

    pos = pos_d / (tnorm * cnorm) * inv_temp
    negs = nd / (nnorm * tnorm) * inv_temp

    m = jnp.maximum(pos, jnp.max(negs, axis=0, keepdims=True))
    lse = m + jnp.log(jnp.exp(pos - m)
                      + jnp.sum(jnp.exp(negs - m), axis=0, keepdims=True))
    part = jnp.sum(lse - pos) * (1.0 / _BATCH)

    @pl.when(i == 0)
    def _():
        o_ref[0, 0] = 0.0

    o_ref[0, 0] += part


def _tc_loss(partials, log_temperature):
    bs = 2048
    grid = _BATCH // bs
    out = pl.pallas_call(
        _tc_loss_body,
        grid=(grid,),
        in_specs=[
            pl.BlockSpec((_NROWS, bs), lambda i: (0, i)),
            pl.BlockSpec(memory_space=pltpu.SMEM),
        ],
        out_specs=pl.BlockSpec(memory_space=pltpu.SMEM),
        out_shape=jax.ShapeDtypeStruct((1, 1), jnp.float32),
    )(partials, jnp.reshape(log_temperature, (1,)))
    return out[0, 0]


def kernel(target_words, context_words, neg_words, in_emb, out_emb,
           log_temperature):
    tw = target_words.astype(jnp.int32)
    cw = context_words.astype(jnp.int32)
    nwf = jnp.reshape(neg_words.astype(jnp.int32), (-1,))

    trows = jnp.take(in_emb, tw, axis=0)
    partials = _sc_dots(trows, out_emb, cw, nwf)
    return _tc_loss(partials, log_temperature)


# speedup vs baseline: 1.6796x; 1.0024x over previous
import functools

import jax
import jax.numpy as jnp
from jax import lax
from jax.experimental import pallas as pl
from jax.experimental.pallas import tpu as pltpu
from jax.experimental.pallas import tpu_sc as plsc

_VOCAB = 1000000
_DIM = 64
_BATCH = 16384
_NEG = 20

_BG = 16
_RPB = _BG * (1 + _NEG)

_ROW_POS = 0
_ROW_TN = 1
_ROW_CN = 2
_ROW_ND = 8
_ROW_NN = 32
_NROWS = 64


def _sc_dots(trows, out_emb, cw, nwf):
    info = plsc.get_sparse_core_info()
    nworkers = info.num_cores * info.num_subcores
    b_per_w = _BATCH // nworkers
    nblk = b_per_w // _BG

    mesh = plsc.VectorSubcoreMesh(core_axis_name="c", subcore_axis_name="s")

    @functools.partial(
        pl.kernel,
        mesh=mesh,
        compiler_params=pltpu.CompilerParams(needs_layout_passes=False),
        out_type=jax.ShapeDtypeStruct((_NROWS, _BATCH), jnp.float32),
        scratch_types=[
            pltpu.VMEM((2 * _RPB,), jnp.int32),
            pltpu.VMEM((2, _RPB, _DIM), jnp.float32),
            pltpu.VMEM((2, _BG, _DIM), jnp.float32),
            pltpu.VMEM((_NROWS, b_per_w), jnp.float32),
            pltpu.SemaphoreType.DMA,
            pltpu.SemaphoreType.DMA,
            pltpu.SemaphoreType.DMA,
            pltpu.SemaphoreType.DMA,
        ],
    )
    def k(t_hbm, out_hbm, cw_hbm, nw_hbm, out_p,
          idx_v, buf, tcol, stage, isem, gsem, tsem, osem):
        wid = lax.axis_index("s") * info.num_cores + lax.axis_index("c")
        b0w = wid * b_per_w

        lane = lax.iota(jnp.int32, 16)
        m15 = lane == 15

        def issue_idx(blk, slot):
            b0 = b0w + blk * _BG
            off = slot * _RPB
            pltpu.async_copy(cw_hbm.at[pl.ds(b0, _BG)],
                             idx_v.at[pl.ds(off, _BG)], isem)
            pltpu.async_copy(nw_hbm.at[pl.ds(b0 * _NEG, _BG * _NEG)],
                             idx_v.at[pl.ds(off + _BG, _BG * _NEG)],
                             isem)

        def wait_idx():
            pltpu.make_async_copy(cw_hbm.at[pl.ds(0, _RPB)],
                                  idx_v.at[pl.ds(0, _RPB)], isem).wait()

        def issue_rows(blk, slot):
            b0 = b0w + blk * _BG
            pltpu.async_copy(t_hbm.at[pl.ds(b0, _BG)], tcol.at[slot], tsem)

            def group(g, _):
                idxv = idx_v[pl.ds(slot * _RPB + g * 16, 16)]
                for r in range(16):
                    pltpu.async_copy(
                        out_hbm.at[pl.ds(idxv[r], 1)],
                        buf.at[slot, pl.ds(g * 16 + r, 1)], gsem)
                return 0

            lax.fori_loop(0, _RPB // 16, group, 0)

        def wait_rows():
            pltpu.make_async_copy(t_hbm.at[pl.ds(0, _BG)],
                                  tcol.at[0], tsem).wait()
            pltpu.make_async_copy(out_hbm.at[pl.ds(0, _RPB)],
                                  buf.at[0], gsem).wait()

        def dot4(a, b):
            s = a[0] * b[0] + a[1] * b[1]
            s = s + a[2] * b[2] + a[3] * b[3]
            return plsc.cumsum(s)

        def put(row, p, v):
            plsc.store_scatter(stage,
                               [jnp.full((16,), row, jnp.int32),
                                jnp.full((16,), p, jnp.int32)],
                               v, mask=m15)

        def compute(blk, slot):
            def body(lb, _):
                p = blk * _BG + lb
                t = [tcol[slot, lb, pl.ds(j * 16, 16)] for j in range(4)]
                c = [buf[slot, lb, pl.ds(j * 16, 16)] for j in range(4)]
                put(_ROW_TN, p, dot4(t, t))
                put(_ROW_CN, p, dot4(c, c))
                put(_ROW_POS, p, dot4(t, c))
                for kk in range(_NEG):
                    row = _BG + lb * _NEG + kk
                    n = [buf[slot, row, pl.ds(j * 16, 16)]
                         for j in range(4)]
                    put(_ROW_ND + kk, p, dot4(n, t))
                    put(_ROW_NN + kk, p, dot4(n, n))
                return 0

            lax.fori_loop(0, _BG, body, 0)

        issue_idx(0, 0)
        wait_idx()
        issue_rows(0, 0)
        issue_idx(1, 1)

        def blk_body(blk, _):
            slot = lax.rem(blk, 2)
            nslot = lax.rem(blk + 1, 2)

            wait_rows()

            @pl.when(blk + 1 < nblk)
            def _():
                wait_idx()
                issue_rows(blk + 1, nslot)

            @pl.when(blk + 2 < nblk)
            def _():
                issue_idx(blk + 2, slot)

            compute(blk, slot)
            return 0

        lax.fori_loop(0, nblk, blk_body, 0)

        pltpu.async_copy(stage, out_p.at[:, pl.ds(b0w, b_per_w)],
                         osem).wait()

    return k(trows, out_emb, cw, nwf)


def _tc_loss_body(x_ref, lt_ref, o_ref):
    i = pl.program_id(0)
    inv_temp = jnp.exp(-lt_ref[0])

    pos_d = x_ref[_ROW_POS:_ROW_POS + 1, :]
    tn = x_ref[_ROW_TN:_ROW_TN + 1, :]
    cn = x_ref[_ROW_CN:_ROW_CN + 1, :]
    nd = x_ref[_ROW_ND:_ROW_ND + _NEG, :]
    nn = x_ref[_ROW_NN:_ROW_NN + _NEG, :]

    tnorm = jnp.maximum(jnp.sqrt(tn), 1e-12)
    cnorm = jnp.maximum(jnp.sqrt(cn), 1e-12)
    nnorm = jnp.maximum(jnp.sqrt(nn), 1e-12)

    pos = pos_d / (tnorm * cnorm) * inv_temp
    negs = nd / (nnorm * tnorm) * inv_temp

    m = jnp.maximum(pos, jnp.max(negs, axis=0, keepdims=True))
    lse = m + jnp.log(jnp.exp(pos - m)
                      + jnp.sum(jnp.exp(negs - m), axis=0, keepdims=True))
    part = jnp.sum(lse - pos) * (1.0 / _BATCH)

    @pl.when(i == 0)
    def _():
        o_ref[0, 0] = 0.0

    o_ref[0, 0] += part


def _tc_loss(partials, log_temperature):
    bs = 2048
    grid = _BATCH // bs
    out = pl.pallas_call(
        _tc_loss_body,
        grid=(grid,),
        in_specs=[
            pl.BlockSpec((_NROWS, bs), lambda i: (0, i)),
            pl.BlockSpec(memory_space=pltpu.SMEM),
        ],
        out_specs=pl.BlockSpec(memory_space=pltpu.SMEM),
        out_shape=jax.ShapeDtypeStruct((1, 1), jnp.float32),
    )(partials, jnp.reshape(log_temperature, (1,)))
    return out[0, 0]


def kernel(target_words, context_words, neg_words, in_emb, out_emb,
           log_temperature):
    tw = target_words.astype(jnp.int32)
    cw = context_words.astype(jnp.int32)
    nwf = jnp.reshape(neg_words.astype(jnp.int32), (-1,))

    in_t = jnp.swapaxes(in_emb, 0, 1)
    trows = jnp.swapaxes(jnp.take(in_t, tw, axis=1), 0, 1)
    partials = _sc_dots(trows, out_emb, cw, nwf)
    return _tc_loss(partials, log_temperature)
